# Initial kernel scaffold; baseline (speedup 1.0000x reference)
import jax
import jax.numpy as jnp
from jax.experimental import pallas as pl


def kernel(x, edge_index, W1, a_s1, a_d1, b1, W2, a_s2, a_d2, b2, lw1, lb1, lw2, lb2):
    raise NotImplementedError("write your pallas kernel here")


# speedup vs baseline: 9.0533x; 9.0533x over previous
import functools

import jax
import jax.numpy as jnp
from jax.experimental import pallas as pl
from jax.experimental.pallas import tpu as pltpu

N = 10000
E = 320000
D = 128
HEADS = 16
OUT = 128

_ROWS = 400


def _dense_body(x_ref, W1_ref, A1s_ref, A1d_ref, W2_ref, A2s_ref, A2d_ref,
                lw1_ref, lb1_ref, lw2_ref, lb2_ref,
                h1_ref, as1_ref, ad1_ref, h2_ref, as2_ref, ad2_ref, xs_ref):
    x = x_ref[...]
    h1 = jax.lax.dot_general(x, W1_ref[...], (((1,), (1,)), ((), ())),
                             preferred_element_type=jnp.float32)
    h2 = jax.lax.dot_general(x, W2_ref[...], (((1,), (1,)), ((), ())),
                             preferred_element_type=jnp.float32)
    h1_ref[...] = h1
    h2_ref[...] = h2
    as1_ref[...] = jnp.dot(h1, A1s_ref[...], preferred_element_type=jnp.float32)
    ad1_ref[...] = jnp.dot(h1, A1d_ref[...], preferred_element_type=jnp.float32)
    as2_ref[...] = jnp.dot(h2, A2s_ref[...], preferred_element_type=jnp.float32)
    ad2_ref[...] = jnp.dot(h2, A2d_ref[...], preferred_element_type=jnp.float32)
    x1 = jax.lax.dot_general(x, lw1_ref[...], (((1,), (1,)), ((), ())),
                             preferred_element_type=jnp.float32) + lb1_ref[...]
    x1 = jnp.where(x1 > 0, x1, jnp.exp(x1) - 1.0)
    xs = jax.lax.dot_general(x1, lw2_ref[...], (((1,), (1,)), ((), ())),
                             preferred_element_type=jnp.float32) + lb2_ref[...]
    xs_ref[...] = jnp.where(xs > 0, xs, jnp.exp(xs) - 1.0)


def _dense_stage(x, W1, A1s, A1d, W2, A2s, A2d, lw1, lb1, lw2, lb2):
    grid = N // _ROWS
    row = lambda i: (i, 0)
    full = lambda i: (0, 0)
    return pl.pallas_call(
        _dense_body,
        grid=(grid,),
        in_specs=[
            pl.BlockSpec((_ROWS, D), row),
            pl.BlockSpec((HEADS * OUT, D), full),
            pl.BlockSpec((HEADS * OUT, HEADS), full),
            pl.BlockSpec((HEADS * OUT, HEADS), full),
            pl.BlockSpec((HEADS * OUT, D), full),
            pl.BlockSpec((HEADS * OUT, HEADS), full),
            pl.BlockSpec((HEADS * OUT, HEADS), full),
            pl.BlockSpec((4 * OUT, D), full),
            pl.BlockSpec((1, 4 * OUT), full),
            pl.BlockSpec((OUT, 4 * OUT), full),
            pl.BlockSpec((1, OUT), full),
        ],
        out_specs=[
            pl.BlockSpec((_ROWS, HEADS * OUT), row),
            pl.BlockSpec((_ROWS, HEADS), row),
            pl.BlockSpec((_ROWS, HEADS), row),
            pl.BlockSpec((_ROWS, HEADS * OUT), row),
            pl.BlockSpec((_ROWS, HEADS), row),
            pl.BlockSpec((_ROWS, HEADS), row),
            pl.BlockSpec((_ROWS, OUT), row),
        ],
        out_shape=[
            jax.ShapeDtypeStruct((N, HEADS * OUT), jnp.float32),
            jax.ShapeDtypeStruct((N, HEADS), jnp.float32),
            jax.ShapeDtypeStruct((N, HEADS), jnp.float32),
            jax.ShapeDtypeStruct((N, HEADS * OUT), jnp.float32),
            jax.ShapeDtypeStruct((N, HEADS), jnp.float32),
            jax.ShapeDtypeStruct((N, HEADS), jnp.float32),
            jax.ShapeDtypeStruct((N, OUT), jnp.float32),
        ],
    )(x, W1, A1s, A1d, W2, A2s, A2d, lw1, lb1, lw2, lb2)


def _edge_stage_xla(h, alpha_src, alpha_dst, src, dst):
    e = alpha_src[src] + alpha_dst[dst]
    e = jnp.where(e > 0, e, 0.2 * e)
    w = jnp.exp(e)
    denom = jax.ops.segment_sum(w, dst, num_segments=N)
    alpha = w / (denom[dst] + 1e-16)
    hr = h.reshape(N, HEADS, OUT)
    msg = jnp.einsum("eh,eho->eo", alpha, hr[src])
    out = jax.ops.segment_sum(msg, dst, num_segments=N)
    return out * (1.0 / HEADS)


def kernel(x, edge_index, W1, a_s1, a_d1, b1, W2, a_s2, a_d2, b2,
           lw1, lb1, lw2, lb2):
    eye = jnp.eye(HEADS, dtype=jnp.float32)
    mk = lambda a: (a[0][:, :, None] * eye[:, None, :]).reshape(HEADS * OUT, HEADS)
    A1s, A1d, A2s, A2d = mk(a_s1), mk(a_d1), mk(a_s2), mk(a_d2)

    h1, as1, ad1, h2, as2, ad2, x_self = _dense_stage(
        x, W1, A1s, A1d, W2, A2s, A2d,
        lw1, lb1.reshape(1, -1), lw2, lb2.reshape(1, -1))

    src, dst = edge_index[0], edge_index[1]
    agg1 = _edge_stage_xla(h1, as1, ad1, src, dst)
    agg2 = _edge_stage_xla(h2, as2, ad2, dst, src)

    x_in = jax.nn.elu(agg1 + b1)
    x_out = jax.nn.elu(agg2 + b2)
    return (x_in, x_out, x_self)


# speedup vs baseline: 15.4253x; 1.7038x over previous
import functools

import jax
import jax.numpy as jnp
from jax import lax
from jax.experimental import pallas as pl
from jax.experimental.pallas import tpu as pltpu
from jax.experimental.pallas import tpu_sc as plsc

N = 10000
E = 320000
D = 128
HEADS = 16
OUT = 128

NC = 2
NS = 16
NW = NC * NS
EPW = E // NW
NPAD = 10240
ROWS_PT = NPAD // NS

_ROWS = 400

C1 = 1000
C1S = 125
C1_SUB = C1 // C1S
C1_CHUNKS = EPW // C1

C2 = 40
C2_CHUNKS = EPW // C2

@functools.lru_cache(maxsize=None)
def _mesh():
    return plsc.VectorSubcoreMesh(core_axis_name="c", subcore_axis_name="s")


def _dense_body(x_ref, W1lo_ref, W1hi_ref, A1s_ref, A1d_ref,
                W2lo_ref, W2hi_ref, A2s_ref, A2d_ref,
                lw1_ref, lb1_ref, lw2_ref, lb2_ref,
                h1lo_ref, h1hi_ref, as1_ref, ad1_ref,
                h2lo_ref, h2hi_ref, as2_ref, ad2_ref, xs_ref):
    x = x_ref[...]
    mm = lambda a, b: lax.dot_general(a, b, (((1,), (1,)), ((), ())),
                                      preferred_element_type=jnp.float32)
    h1lo = mm(x, W1lo_ref[...])
    h1hi = mm(x, W1hi_ref[...])
    h2lo = mm(x, W2lo_ref[...])
    h2hi = mm(x, W2hi_ref[...])
    h1lo_ref[...] = h1lo
    h1hi_ref[...] = h1hi
    h2lo_ref[...] = h2lo
    h2hi_ref[...] = h2hi
    h1 = jnp.concatenate([h1lo, h1hi], axis=1)
    h2 = jnp.concatenate([h2lo, h2hi], axis=1)
    as1_ref[...] = jnp.dot(h1, A1s_ref[...], preferred_element_type=jnp.float32)
    ad1_ref[...] = jnp.dot(h1, A1d_ref[...], preferred_element_type=jnp.float32)
    as2_ref[...] = jnp.dot(h2, A2s_ref[...], preferred_element_type=jnp.float32)
    ad2_ref[...] = jnp.dot(h2, A2d_ref[...], preferred_element_type=jnp.float32)
    x1 = lax.dot_general(x, lw1_ref[...], (((1,), (1,)), ((), ())),
                         preferred_element_type=jnp.float32) + lb1_ref[...]
    x1 = jnp.where(x1 > 0, x1, jnp.exp(x1) - 1.0)
    xs = lax.dot_general(x1, lw2_ref[...], (((1,), (1,)), ((), ())),
                         preferred_element_type=jnp.float32) + lb2_ref[...]
    xs_ref[...] = jnp.where(xs > 0, xs, jnp.exp(xs) - 1.0)


HALF = HEADS * OUT // 2


def _dense_stage(x, W1lo, W1hi, A1s, A1d, W2lo, W2hi, A2s, A2d,
                 lw1, lb1, lw2, lb2):
    grid = N // _ROWS
    row = lambda i: (i, 0)
    full = lambda i: (0, 0)
    return pl.pallas_call(
        _dense_body,
        grid=(grid,),
        in_specs=[
            pl.BlockSpec((_ROWS, D), row),
            pl.BlockSpec((HALF, D), full),
            pl.BlockSpec((HALF, D), full),
            pl.BlockSpec((HEADS * OUT, HEADS), full),
            pl.BlockSpec((HEADS * OUT, HEADS), full),
            pl.BlockSpec((HALF, D), full),
            pl.BlockSpec((HALF, D), full),
            pl.BlockSpec((HEADS * OUT, HEADS), full),
            pl.BlockSpec((HEADS * OUT, HEADS), full),
            pl.BlockSpec((4 * OUT, D), full),
            pl.BlockSpec((1, 4 * OUT), full),
            pl.BlockSpec((OUT, 4 * OUT), full),
            pl.BlockSpec((1, OUT), full),
        ],
        out_specs=[
            pl.BlockSpec((_ROWS, HALF), row),
            pl.BlockSpec((_ROWS, HALF), row),
            pl.BlockSpec((_ROWS, HEADS), row),
            pl.BlockSpec((_ROWS, HEADS), row),
            pl.BlockSpec((_ROWS, HALF), row),
            pl.BlockSpec((_ROWS, HALF), row),
            pl.BlockSpec((_ROWS, HEADS), row),
            pl.BlockSpec((_ROWS, HEADS), row),
            pl.BlockSpec((_ROWS, OUT), row),
        ],
        out_shape=[
            jax.ShapeDtypeStruct((N, HALF), jnp.float32),
            jax.ShapeDtypeStruct((N, HALF), jnp.float32),
            jax.ShapeDtypeStruct((N, HEADS), jnp.float32),
            jax.ShapeDtypeStruct((N, HEADS), jnp.float32),
            jax.ShapeDtypeStruct((N, HALF), jnp.float32),
            jax.ShapeDtypeStruct((N, HALF), jnp.float32),
            jax.ShapeDtypeStruct((N, HEADS), jnp.float32),
            jax.ShapeDtypeStruct((N, HEADS), jnp.float32),
            jax.ShapeDtypeStruct((N, OUT), jnp.float32),
        ],
    )(x, W1lo, W1hi, A1s, A1d, W2lo, W2hi, A2s, A2d, lw1, lb1, lw2, lb2)


def _pass1_body(src_hbm, dst_hbm, asrc_hbm, adst_hbm, z16_hbm,
                w_hbm, dpart_hbm,
                idx_s, idx_d, as_v, ad_v, w_v, dacc, sem):
    c = lax.axis_index("c")
    s = lax.axis_index("s")
    wid = s * NC + c

    pltpu.sync_copy(z16_hbm.at[pl.ds(s * ROWS_PT, ROWS_PT)],
                    dacc.at[pl.ds(s * ROWS_PT, ROWS_PT)])
    plsc.subcore_barrier()

    def chunk(ci, carry):
        row0 = wid * (EPW // C1S) + ci * C1_SUB
        off = wid * EPW + ci * C1
        pltpu.sync_copy(src_hbm.at[pl.ds(row0, C1_SUB)], idx_s)
        pltpu.sync_copy(dst_hbm.at[pl.ds(row0, C1_SUB)], idx_d)
        for j in range(C1_SUB):
            pltpu.async_copy(asrc_hbm.at[idx_s.at[j]],
                             as_v.at[pl.ds(j * C1S, C1S)], sem).wait()
            pltpu.async_copy(adst_hbm.at[idx_d.at[j]],
                             ad_v.at[pl.ds(j * C1S, C1S)], sem).wait()

        def body(i, carry2):
            e = as_v[i, :] + ad_v[i, :]
            e = jnp.where(e > 0.0, e, 0.2 * e)
            w_v[i, :] = jnp.exp(e)
            return carry2
        lax.fori_loop(0, C1, body, 0)

        for j in range(C1_SUB):
            pltpu.sync_copy(w_v.at[pl.ds(j * C1S, C1S)],
                            dacc.at[idx_d.at[j]], add=True)
        pltpu.sync_copy(w_v, w_hbm.at[pl.ds(off, C1)])
        return carry
    lax.fori_loop(0, C1_CHUNKS, chunk, 0)

    plsc.subcore_barrier()
    pltpu.sync_copy(dacc.at[pl.ds(s * ROWS_PT, ROWS_PT)],
                    dpart_hbm.at[c, pl.ds(s * ROWS_PT, ROWS_PT)])


@functools.lru_cache(maxsize=None)
def _pass1():
    return pl.kernel(
        _pass1_body,
        out_type=[jax.ShapeDtypeStruct((E, HEADS), jnp.float32),
                  jax.ShapeDtypeStruct((NC, NPAD, HEADS), jnp.float32)],
        mesh=_mesh(),
        compiler_params=pltpu.CompilerParams(use_tc_tiling_on_sc=False),
        scratch_types=[
        pltpu.VMEM((C1_SUB, C1S), jnp.int32),
        pltpu.VMEM((C1_SUB, C1S), jnp.int32),
        pltpu.VMEM((C1, HEADS), jnp.float32),
        pltpu.VMEM((C1, HEADS), jnp.float32),
            pltpu.VMEM((C1, HEADS), jnp.float32),
            pltpu.VMEM_SHARED((NPAD, HEADS), jnp.float32),
            pltpu.SemaphoreType.DMA,
        ],
    )


def _pass2_body(src_hbm, dst_hbm, w_hbm, invd_hbm, h_hbm, z128_hbm,
                out_hbm,
                idx_sv, idx_dv, h_rows, w_v, invd_v, alpha_v, msg, oacc,
                sem_h, sem_g):
    c = lax.axis_index("c")
    s = lax.axis_index("s")
    wid = s * NC + c

    pltpu.sync_copy(z128_hbm.at[pl.ds(s * ROWS_PT, ROWS_PT)],
                    oacc.at[pl.ds(s * ROWS_PT, ROWS_PT)])
    plsc.subcore_barrier()

    def chunk(ci, carry):
        off = wid * EPW + ci * C2
        pltpu.sync_copy(src_hbm.at[pl.ds(off, C2)], idx_sv)
        pltpu.sync_copy(dst_hbm.at[pl.ds(off, C2)], idx_dv)
        cp_h = pltpu.async_copy(h_hbm.at[idx_sv], h_rows, sem_h)
        cp_g = pltpu.async_copy(invd_hbm.at[idx_dv], invd_v, sem_g)
        pltpu.sync_copy(w_hbm.at[pl.ds(off, C2)], w_v)
        cp_g.wait()

        def abody(i, carry2):
            alpha_v[i, :] = w_v[i, :] * invd_v[i, :]
            return carry2
        lax.fori_loop(0, C2, abody, 0)
        cp_h.wait()

        def mbody(i, carry2):
            accs = [jnp.zeros((16,), jnp.float32) for _ in range(4)]
            arow = alpha_v[i, :]
            for hd in range(HEADS):
                a = arow[hd]
                for j2 in range(4):
                    accs[j2] = accs[j2] + a * h_rows[i, pl.ds(hd * 64 + j2 * 16, 16)]
            for j2 in range(4):
                msg[i, pl.ds(j2 * 16, 16)] = accs[j2]
            return carry2
        lax.fori_loop(0, C2, mbody, 0)

        pltpu.sync_copy(msg, oacc.at[idx_dv], add=True)
        return carry
    lax.fori_loop(0, C2_CHUNKS, chunk, 0)

    plsc.subcore_barrier()
    pltpu.sync_copy(oacc.at[pl.ds(s * ROWS_PT, ROWS_PT)],
                    out_hbm.at[c, pl.ds(s * ROWS_PT, ROWS_PT)])


@functools.lru_cache(maxsize=None)
def _pass2():
    return pl.kernel(
        _pass2_body,
        out_type=jax.ShapeDtypeStruct((NC, NPAD, OUT // 2), jnp.float32),
        mesh=_mesh(),
        compiler_params=pltpu.CompilerParams(use_tc_tiling_on_sc=False),
        scratch_types=[
            pltpu.VMEM((C2,), jnp.int32),
            pltpu.VMEM((C2,), jnp.int32),
            pltpu.VMEM((C2, HALF), jnp.float32),
            pltpu.VMEM((C2, HEADS), jnp.float32),
            pltpu.VMEM((C2, HEADS), jnp.float32),
            pltpu.VMEM((C2, HEADS), jnp.float32),
            pltpu.VMEM((C2, OUT // 2), jnp.float32),
            pltpu.VMEM_SHARED((NPAD, OUT // 2), jnp.float32),
            pltpu.SemaphoreType.DMA,
            pltpu.SemaphoreType.DMA,
        ],
    )


def _invd_body(dpart_ref, invd_ref):
    d = dpart_ref[...]
    invd_ref[...] = 1.0 / (d[0] + d[1] + 1e-16)


def _invd_stage(dpart):
    return pl.pallas_call(
        _invd_body,
        in_specs=[pl.BlockSpec((NC, NPAD, HEADS), lambda: (0, 0, 0))],
        out_specs=pl.BlockSpec((NPAD, HEADS), lambda: (0, 0)),
        out_shape=jax.ShapeDtypeStruct((NPAD, HEADS), jnp.float32),
    )(dpart)


def _final_body(o1lo_ref, o1hi_ref, o2lo_ref, o2hi_ref, b1_ref, b2_ref,
                xin_ref, xout_ref):
    o1 = jnp.concatenate([o1lo_ref[0] + o1lo_ref[1],
                          o1hi_ref[0] + o1hi_ref[1]], axis=1)
    o1 = o1 * (1.0 / HEADS) + b1_ref[...]
    xin_ref[...] = jnp.where(o1 > 0, o1, jnp.exp(o1) - 1.0)
    o2 = jnp.concatenate([o2lo_ref[0] + o2lo_ref[1],
                          o2hi_ref[0] + o2hi_ref[1]], axis=1)
    o2 = o2 * (1.0 / HEADS) + b2_ref[...]
    xout_ref[...] = jnp.where(o2 > 0, o2, jnp.exp(o2) - 1.0)


def _final_stage(o1lo, o1hi, o2lo, o2hi, b1, b2):
    grid = N // _ROWS
    return pl.pallas_call(
        _final_body,
        grid=(grid,),
        in_specs=[
            pl.BlockSpec((NC, _ROWS, OUT // 2), lambda i: (0, i, 0)),
            pl.BlockSpec((NC, _ROWS, OUT // 2), lambda i: (0, i, 0)),
            pl.BlockSpec((NC, _ROWS, OUT // 2), lambda i: (0, i, 0)),
            pl.BlockSpec((NC, _ROWS, OUT // 2), lambda i: (0, i, 0)),
            pl.BlockSpec((1, OUT), lambda i: (0, 0)),
            pl.BlockSpec((1, OUT), lambda i: (0, 0)),
        ],
        out_specs=[
            pl.BlockSpec((_ROWS, OUT), lambda i: (i, 0)),
            pl.BlockSpec((_ROWS, OUT), lambda i: (i, 0)),
        ],
        out_shape=[
            jax.ShapeDtypeStruct((N, OUT), jnp.float32),
            jax.ShapeDtypeStruct((N, OUT), jnp.float32),
        ],
    )(o1lo, o1hi, o2lo, o2hi, b1, b2)


def _edge_stage_sc(h_lo, h_hi, alpha_src, alpha_dst, src, dst):
    src1 = src.reshape(E // C1S, C1S)
    dst1 = dst.reshape(E // C1S, C1S)
    z16 = jnp.zeros((NPAD, HEADS), jnp.float32)
    w, dpart = _pass1()(src1, dst1, alpha_src, alpha_dst, z16)
    invd = _invd_stage(dpart)
    z64 = jnp.zeros((NPAD, OUT // 2), jnp.float32)
    o_lo = _pass2()(src, dst, w, invd, h_lo, z64)
    o_hi = _pass2()(src, dst, w, invd, h_hi, z64)
    return o_lo, o_hi


def kernel(x, edge_index, W1, a_s1, a_d1, b1, W2, a_s2, a_d2, b2,
           lw1, lb1, lw2, lb2):
    eye = jnp.eye(HEADS, dtype=jnp.float32)

    def bd(a2):
        return (a2[:, :, None] * eye[:, None, :]).reshape(HALF, HEADS)

    def mk(a):
        return jnp.concatenate([bd(a[0][:, :OUT // 2]), bd(a[0][:, OUT // 2:])],
                               axis=0)

    def wsplit(W):
        Wr = W.reshape(HEADS, 2, OUT // 2, D)
        return (Wr[:, 0].reshape(HALF, D), Wr[:, 1].reshape(HALF, D))

    A1s, A1d, A2s, A2d = mk(a_s1), mk(a_d1), mk(a_s2), mk(a_d2)
    W1lo, W1hi = wsplit(W1)
    W2lo, W2hi = wsplit(W2)

    (h1lo, h1hi, as1, ad1, h2lo, h2hi, as2, ad2, x_self) = _dense_stage(
        x, W1lo, W1hi, A1s, A1d, W2lo, W2hi, A2s, A2d,
        lw1, lb1.reshape(1, -1), lw2, lb2.reshape(1, -1))

    src, dst = edge_index[0], edge_index[1]
    o1lo, o1hi = _edge_stage_sc(h1lo, h1hi, as1, ad1, src, dst)
    o2lo, o2hi = _edge_stage_sc(h2lo, h2hi, as2, ad2, dst, src)

    x_in, x_out = _final_stage(o1lo, o1hi, o2lo, o2hi,
                               b1.reshape(1, -1), b2.reshape(1, -1))
    return (x_in, x_out, x_self)
